# baseline (speedup 1.0000x reference)
import functools

import jax
import jax.numpy as jnp
from jax import lax
from jax.experimental import pallas as pl
from jax.experimental.pallas import tpu as pltpu
from jax.experimental.pallas import tpu_sc as plsc

_B = 64
_S = 1025
_L = 1024
_K = 512
_D = 768
_OUT_S = 1 + _K
_IDX_PAD = 520
_CHUNK = 64
_NCH_B = 8
_TAIL = 512


def _gather_body(x_hbm, idx_hbm, out_hbm, idx_v, buf0, buf1, tbuf,
                 semg0, semg1, sems0, sems1):
    wid = lax.axis_index("s") * 2 + lax.axis_index("c")

    batches = (wid * 2, wid * 2 + 1)
    for bi, b in enumerate(batches):
        pltpu.sync_copy(
            idx_hbm.at[pl.ds(b * _IDX_PAD, _IDX_PAD)],
            idx_v.at[pl.ds(bi * _IDX_PAD, _IDX_PAD)],
        )

    bufs = (buf0, buf1)
    semg = (semg0, semg1)
    sems = (sems0, sems1)
    work = [(bi, c * _CHUNK) for bi in range(2) for c in range(_NCH_B)]
    n = len(work)

    def gather(i):
        bi, off = work[i]
        k = i % 2
        idx_slice = idx_v.at[pl.ds(bi * _IDX_PAD + off, _CHUNK)]
        return pltpu.async_copy(
            x_hbm.at[batches[bi]].at[idx_slice], bufs[k], semg[k],
        )

    def store(i):
        bi, off = work[i]
        k = i % 2
        return pltpu.async_copy(
            bufs[k], out_hbm.at[batches[bi]].at[pl.ds(off, _CHUNK)], sems[k],
        )

    g_d = gather(0)
    s_prev = None
    for i in range(n):
        g_d.wait()
        s_d = store(i)
        if s_prev is not None:
            s_prev.wait()
        if i + 1 < n:
            g_d = gather(i + 1)
        s_prev = s_d
    s_prev.wait()

    for bi in range(2):
        idx_slice = idx_v.at[pl.ds(bi * _IDX_PAD + _TAIL, 1)]
        pltpu.async_copy(
            x_hbm.at[batches[bi]].at[idx_slice], tbuf, semg0
        ).wait()
        pltpu.sync_copy(tbuf, out_hbm.at[batches[bi]].at[pl.ds(_TAIL, 1)])


@functools.partial(
    pl.kernel,
    mesh=plsc.VectorSubcoreMesh(core_axis_name="c", subcore_axis_name="s"),
    out_type=jax.ShapeDtypeStruct((_B, _OUT_S, _D), jnp.float32),
    scratch_types=[
        pltpu.VMEM((2 * _IDX_PAD,), jnp.int32),
        pltpu.VMEM((_CHUNK, _D), jnp.float32),
        pltpu.VMEM((_CHUNK, _D), jnp.float32),
        pltpu.VMEM((1, _D), jnp.float32),
        pltpu.SemaphoreType.DMA,
        pltpu.SemaphoreType.DMA,
        pltpu.SemaphoreType.DMA,
        pltpu.SemaphoreType.DMA,
    ],
)
def _sc_gather(*refs):
    _gather_body(*refs)


def _row_indices():
    noise = jax.random.normal(jax.random.key(1), (_B, _L), dtype=jnp.float32)
    keep = jnp.argsort(noise, axis=-1)[:, :_K].astype(jnp.int32)
    rows = jnp.concatenate(
        [jnp.zeros((_B, 1), jnp.int32), keep + 1], axis=1
    )
    pad = jnp.zeros((_B, _IDX_PAD - _OUT_S), jnp.int32)
    return jnp.concatenate([rows, pad], axis=1).reshape(_B * _IDX_PAD)


def kernel(x):
    return _sc_gather(x, _row_indices())


# speedup vs baseline: 1.0010x; 1.0010x over previous
import functools

import jax
import jax.numpy as jnp
from jax import lax
from jax.experimental import pallas as pl
from jax.experimental.pallas import tpu as pltpu
from jax.experimental.pallas import tpu_sc as plsc

_B = 64
_S = 1025
_L = 1024
_K = 512
_D = 768
_OUT_S = 1 + _K
_IDX_PAD = 520
_CHUNK = 64
_NCH_B = 8
_TAIL = 512


def _gather_body(x_hbm, idx_hbm, out_hbm, idx_v, buf0, buf1, tbuf,
                 semg0, semg1, sems0, sems1):
    wid = lax.axis_index("s") * 2 + lax.axis_index("c")

    batches = (wid * 2, wid * 2 + 1)
    for bi, b in enumerate(batches):
        pltpu.sync_copy(
            idx_hbm.at[pl.ds(b * _IDX_PAD, _IDX_PAD)],
            idx_v.at[pl.ds(bi * _IDX_PAD, _IDX_PAD)],
        )

    bufs = (buf0, buf1)
    semg = (semg0, semg1)
    sems = (sems0, sems1)
    work = [(bi, c * _CHUNK) for bi in range(2) for c in range(_NCH_B)]
    n = len(work)

    def gather(i):
        bi, off = work[i]
        k = i % 2
        idx_slice = idx_v.at[pl.ds(bi * _IDX_PAD + off, _CHUNK)]
        return pltpu.async_copy(
            x_hbm.at[batches[bi]].at[idx_slice], bufs[k], semg[k],
        )

    def store(i):
        bi, off = work[i]
        k = i % 2
        return pltpu.async_copy(
            bufs[k], out_hbm.at[batches[bi]].at[pl.ds(off, _CHUNK)], sems[k],
        )

    g_d = gather(0)
    s_prev = None
    for i in range(n):
        g_d.wait()
        s_d = store(i)
        if s_prev is not None:
            s_prev.wait()
        if i + 1 < n:
            g_d = gather(i + 1)
        s_prev = s_d
    s_prev.wait()

    for bi in range(2):
        idx_slice = idx_v.at[pl.ds(bi * _IDX_PAD + _TAIL, 1)]
        pltpu.async_copy(
            x_hbm.at[batches[bi]].at[idx_slice], tbuf, semg0
        ).wait()
        pltpu.sync_copy(tbuf, out_hbm.at[batches[bi]].at[pl.ds(_TAIL, 1)])


@functools.partial(
    pl.kernel,
    mesh=plsc.VectorSubcoreMesh(core_axis_name="c", subcore_axis_name="s"),
    compiler_params=pltpu.CompilerParams(use_tc_tiling_on_sc=True),
    out_type=jax.ShapeDtypeStruct((_B, _OUT_S, _D), jnp.float32),
    scratch_types=[
        pltpu.VMEM((2 * _IDX_PAD,), jnp.int32),
        pltpu.VMEM((_CHUNK, _D), jnp.float32),
        pltpu.VMEM((_CHUNK, _D), jnp.float32),
        pltpu.VMEM((1, _D), jnp.float32),
        pltpu.SemaphoreType.DMA,
        pltpu.SemaphoreType.DMA,
        pltpu.SemaphoreType.DMA,
        pltpu.SemaphoreType.DMA,
    ],
)
def _sc_gather(*refs):
    _gather_body(*refs)


def _row_indices():
    noise = jax.random.normal(jax.random.key(1), (_B, _L), dtype=jnp.float32)
    keep = jnp.argsort(noise, axis=-1)[:, :_K].astype(jnp.int32)
    rows = jnp.concatenate(
        [jnp.zeros((_B, 1), jnp.int32), keep + 1], axis=1
    )
    pad = jnp.zeros((_B, _IDX_PAD - _OUT_S), jnp.int32)
    return jnp.concatenate([rows, pad], axis=1).reshape(_B * _IDX_PAD)


def kernel(x):
    return _sc_gather(x, _row_indices())


# speedup vs baseline: 2.5354x; 2.5327x over previous
import functools

import jax
import jax.numpy as jnp
from jax import lax
from jax.experimental import pallas as pl
from jax.experimental.pallas import tpu as pltpu
from jax.experimental.pallas import tpu_sc as plsc

_B = 64
_S = 1025
_L = 1024
_K = 512
_D = 768
_OUT_S = 1 + _K
_TOTAL = _B * _OUT_S
_NW = 32
_CHUNK = 64
_NCH_W = 16
_ROWS_W = _NCH_W * _CHUNK
_TAIL_BASE = _NW * _ROWS_W


def _gather_body(table_hbm, idx_hbm, out_hbm, idx_v, idx_t, buf0, buf1,
                 semg0, semg1, sems0, sems1):
    wid = lax.axis_index("s") * 2 + lax.axis_index("c")
    base = wid * _ROWS_W
    pltpu.sync_copy(idx_hbm.at[pl.ds(base, _ROWS_W)], idx_v)

    bufs = (buf0, buf1)
    semg = (semg0, semg1)
    sems = (sems0, sems1)

    def gather(j):
        k = j % 2
        return pltpu.async_copy(
            table_hbm.at[idx_v.at[pl.ds(j * _CHUNK, _CHUNK)]], bufs[k], semg[k]
        )

    def store(j):
        k = j % 2
        return pltpu.async_copy(
            bufs[k], out_hbm.at[pl.ds(base + j * _CHUNK, _CHUNK)], sems[k]
        )

    g_d = gather(0)
    s_prev = None
    for j in range(_NCH_W):
        g_d.wait()
        s_d = store(j)
        if s_prev is not None:
            s_prev.wait()
        if j + 1 < _NCH_W:
            g_d = gather(j + 1)
        s_prev = s_d
    s_prev.wait()

    @pl.when(wid == 0)
    def _():
        pltpu.sync_copy(idx_hbm.at[pl.ds(_TAIL_BASE, _CHUNK)], idx_t)
        pltpu.async_copy(table_hbm.at[idx_t], buf0, semg0).wait()
        pltpu.sync_copy(buf0, out_hbm.at[pl.ds(_TAIL_BASE, _CHUNK)])


@functools.partial(
    pl.kernel,
    mesh=plsc.VectorSubcoreMesh(core_axis_name="c", subcore_axis_name="s"),
    out_type=jax.ShapeDtypeStruct((_TOTAL, _D), jnp.float32),
    scratch_types=[
        pltpu.VMEM((_ROWS_W,), jnp.int32),
        pltpu.VMEM((_CHUNK,), jnp.int32),
        pltpu.VMEM((_CHUNK, _D), jnp.float32),
        pltpu.VMEM((_CHUNK, _D), jnp.float32),
        pltpu.SemaphoreType.DMA,
        pltpu.SemaphoreType.DMA,
        pltpu.SemaphoreType.DMA,
        pltpu.SemaphoreType.DMA,
    ],
)
def _sc_gather(*refs):
    _gather_body(*refs)


def _row_indices():
    noise = jax.random.normal(jax.random.key(1), (_B, _L), dtype=jnp.float32)
    keep = jnp.argsort(noise, axis=-1)[:, :_K].astype(jnp.int32)
    rows = jnp.concatenate(
        [jnp.zeros((_B, 1), jnp.int32), keep + 1], axis=1
    )
    gidx = rows.T * _B + jnp.arange(_B, dtype=jnp.int32)[None, :]
    return gidx.reshape(_TOTAL)


def kernel(x):
    table = x.transpose(1, 0, 2).reshape(_S * _B, _D)
    out_flat = _sc_gather(table, _row_indices())
    return out_flat.reshape(_OUT_S, _B, _D).transpose(1, 0, 2)


# speedup vs baseline: 3.1693x; 1.2500x over previous
import functools

import jax
import jax.numpy as jnp
from jax import lax
from jax.experimental import pallas as pl
from jax.experimental.pallas import tpu as pltpu
from jax.experimental.pallas import tpu_sc as plsc

_B = 64
_S = 1025
_L = 1024
_K = 512
_D = 768
_OUT_S = 1 + _K
_TOTAL = _B * _OUT_S
_NW = 32
_CHUNK = 64
_NCH_W = 16
_ROWS_W = _NCH_W * _CHUNK
_TAIL_BASE = _NW * _ROWS_W


def _gather_body(table_hbm, idx_hbm, out_hbm, idx_v, idx_t, buf0, buf1,
                 semg0, semg1, sems0, sems1):
    wid = lax.axis_index("s") * 2 + lax.axis_index("c")
    base = wid * _ROWS_W
    pltpu.sync_copy(idx_hbm.at[pl.ds(base, _ROWS_W)], idx_v)

    bufs = (buf0, buf1)
    semg = (semg0, semg1)
    sems = (sems0, sems1)

    def gather(j):
        k = j % 2
        return pltpu.async_copy(
            table_hbm.at[idx_v.at[pl.ds(j * _CHUNK, _CHUNK)]], bufs[k], semg[k]
        )

    def store(j):
        k = j % 2
        return pltpu.async_copy(
            bufs[k], out_hbm.at[pl.ds(base + j * _CHUNK, _CHUNK)], sems[k]
        )

    g_d = gather(0)
    s_prev = None
    for j in range(_NCH_W):
        g_d.wait()
        s_d = store(j)
        if s_prev is not None:
            s_prev.wait()
        if j + 1 < _NCH_W:
            g_d = gather(j + 1)
        s_prev = s_d
    s_prev.wait()

    @pl.when(wid == 0)
    def _():
        pltpu.sync_copy(idx_hbm.at[pl.ds(_TAIL_BASE, _CHUNK)], idx_t)
        pltpu.async_copy(table_hbm.at[idx_t], buf0, semg0).wait()
        pltpu.sync_copy(buf0, out_hbm.at[pl.ds(_TAIL_BASE, _CHUNK)])


@functools.partial(
    pl.kernel,
    mesh=plsc.VectorSubcoreMesh(core_axis_name="c", subcore_axis_name="s"),
    out_type=jax.ShapeDtypeStruct((_TOTAL, _D), jnp.float32),
    scratch_types=[
        pltpu.VMEM((_ROWS_W,), jnp.int32),
        pltpu.VMEM((_CHUNK,), jnp.int32),
        pltpu.VMEM((_CHUNK, _D), jnp.float32),
        pltpu.VMEM((_CHUNK, _D), jnp.float32),
        pltpu.SemaphoreType.DMA,
        pltpu.SemaphoreType.DMA,
        pltpu.SemaphoreType.DMA,
        pltpu.SemaphoreType.DMA,
    ],
)
def _sc_gather(*refs):
    _gather_body(*refs)


def _row_indices():
    with jax.ensure_compile_time_eval():
        noise = jax.random.normal(jax.random.key(1), (_B, _L), dtype=jnp.float32)
        keep = jnp.argsort(noise, axis=-1)[:, :_K].astype(jnp.int32)
        rows = jnp.concatenate(
            [jnp.zeros((_B, 1), jnp.int32), keep + 1], axis=1
        )
        gidx = rows.T * _B + jnp.arange(_B, dtype=jnp.int32)[None, :]
        return gidx.reshape(_TOTAL)


def kernel(x):
    table = x.transpose(1, 0, 2).reshape(_S * _B, _D)
    out_flat = _sc_gather(table, _row_indices())
    return out_flat.reshape(_OUT_S, _B, _D).transpose(1, 0, 2)


# speedup vs baseline: 3.2019x; 1.0103x over previous
import functools

import jax
import jax.numpy as jnp
from jax import lax
from jax.experimental import pallas as pl
from jax.experimental.pallas import tpu as pltpu
from jax.experimental.pallas import tpu_sc as plsc

_B = 64
_S = 1025
_L = 1024
_K = 512
_D = 768
_OUT_S = 1 + _K
_TOTAL = _B * _OUT_S
_NW = 32
_CHUNK = 64
_NCH_W = 16
_ROWS_W = _NCH_W * _CHUNK
_TAIL_BASE = _NW * _ROWS_W


_CH = [(c * 48, 48) for c in range(21)] + [(1008, 16)]
_BUF_ROWS = 48


def _gather_body(table_hbm, idx_hbm, out_hbm, idx_v, idx_t, buf0, buf1, buf2,
                 semg0, semg1, semg2, sems0, sems1, sems2):
    wid = lax.axis_index("s") * 2 + lax.axis_index("c")
    base = wid * _ROWS_W
    pltpu.sync_copy(idx_hbm.at[pl.ds(base, _ROWS_W)], idx_v)

    bufs = (buf0, buf1, buf2)
    semg = (semg0, semg1, semg2)
    sems = (sems0, sems1, sems2)
    n = len(_CH)

    def gather(j):
        off, ln = _CH[j]
        k = j % 3
        return pltpu.async_copy(
            table_hbm.at[idx_v.at[pl.ds(off, ln)]], bufs[k].at[pl.ds(0, ln)],
            semg[k],
        )

    def store(j):
        off, ln = _CH[j]
        k = j % 3
        return pltpu.async_copy(
            bufs[k].at[pl.ds(0, ln)], out_hbm.at[pl.ds(base + off, ln)],
            sems[k],
        )

    g_d = {0: gather(0), 1: gather(1)}
    s_d = {}
    for j in range(n):
        g_d[j].wait()
        s_d[j] = store(j)
        if j - 1 >= 0:
            s_d[j - 1].wait()
        if j + 2 < n:
            g_d[j + 2] = gather(j + 2)
    s_d[n - 1].wait()

    @pl.when(wid == 0)
    def _():
        pltpu.sync_copy(idx_hbm.at[pl.ds(_TAIL_BASE, _CHUNK)], idx_t)
        for off, ln in ((0, 48), (48, 16)):
            pltpu.async_copy(
                table_hbm.at[idx_t.at[pl.ds(off, ln)]],
                buf0.at[pl.ds(0, ln)], semg0,
            ).wait()
            pltpu.sync_copy(
                buf0.at[pl.ds(0, ln)],
                out_hbm.at[pl.ds(_TAIL_BASE + off, ln)],
            )


@functools.partial(
    pl.kernel,
    mesh=plsc.VectorSubcoreMesh(core_axis_name="c", subcore_axis_name="s"),
    out_type=jax.ShapeDtypeStruct((_TOTAL, _D), jnp.float32),
    scratch_types=[
        pltpu.VMEM((_ROWS_W,), jnp.int32),
        pltpu.VMEM((_CHUNK,), jnp.int32),
        pltpu.VMEM((_BUF_ROWS, _D), jnp.float32),
        pltpu.VMEM((_BUF_ROWS, _D), jnp.float32),
        pltpu.VMEM((_BUF_ROWS, _D), jnp.float32),
        pltpu.SemaphoreType.DMA,
        pltpu.SemaphoreType.DMA,
        pltpu.SemaphoreType.DMA,
        pltpu.SemaphoreType.DMA,
        pltpu.SemaphoreType.DMA,
        pltpu.SemaphoreType.DMA,
    ],
)
def _sc_gather(*refs):
    _gather_body(*refs)


def _row_indices():
    with jax.ensure_compile_time_eval():
        noise = jax.random.normal(jax.random.key(1), (_B, _L), dtype=jnp.float32)
        keep = jnp.argsort(noise, axis=-1)[:, :_K].astype(jnp.int32)
        rows = jnp.concatenate(
            [jnp.zeros((_B, 1), jnp.int32), keep + 1], axis=1
        )
        gidx = rows.T * _B + jnp.arange(_B, dtype=jnp.int32)[None, :]
        return gidx.reshape(_TOTAL)


def kernel(x):
    table = x.transpose(1, 0, 2).reshape(_S * _B, _D)
    out_flat = _sc_gather(table, _row_indices())
    return out_flat.reshape(_OUT_S, _B, _D).transpose(1, 0, 2)
